# baseline (speedup 1.0000x reference)
import functools

import jax
import jax.numpy as jnp
from jax.experimental import pallas as pl
from jax.experimental.pallas import tpu as pltpu

_LN_EPS = 1e-5


def _ln(v, g, b):
    mu = jnp.mean(v, axis=-1, keepdims=True)
    var = jnp.mean((v - mu) ** 2, axis=-1, keepdims=True)
    return (v - mu) * jax.lax.rsqrt(var + _LN_EPS) * g + b


def _mm(a, b):
    return jnp.dot(a, b, preferred_element_type=jnp.float32)


def _node_kernel(x_ref, idx_ref, wn_ref, wvin_ref, wv_ref, w2_ref, w3_ref,
                 we_ref, w1e_ref, wvine_ref, wve_ref,
                 bn_ref, g1n_ref, be1n_ref, bvinn_ref, bvn_ref, g2n_ref,
                 be2n_ref, b1e_ref, b2e_ref, b3e_ref, be_ref, bvine_b_ref,
                 bve_b_ref,
                 hnode_ref, fw_ref, tw_ref, wce_ref, wvbe_ref, bve_ref):
    B, N, H = x_ref.shape
    K = idx_ref.shape[-1]

    x3 = x_ref[...].astype(jnp.float32)
    idx3 = idx_ref[...]
    x = x3.reshape(B * N, H)

    b_node, g1, be1 = bn_ref[...], g1n_ref[...], be1n_ref[...]
    b_vin, b_v, g2, be2 = bvinn_ref[...], bvn_ref[...], g2n_ref[...], be2n_ref[...]
    b12_e = b1e_ref[...] + b2e_ref[...]
    b3_e, b_e = b3e_ref[...], be_ref[...]
    b_vin_e, b_v_e = bvine_b_ref[...], bve_b_ref[...]

    inv_sqrt_h = jnp.float32(1.0) / jnp.sqrt(jnp.float32(H))
    j_iota = jax.lax.broadcasted_iota(jnp.int32, (N, N), 1)
    weighted = []
    for bi in range(B):
        xb = x3[bi]
        idx = idx3[bi]
        s = jax.lax.dot_general(xb, xb, (((1,), (1,)), ((), ())),
                                preferred_element_type=jnp.float32) * inv_sqrt_h
        counts = jnp.zeros((N, N), jnp.float32)
        for m in range(K):
            counts = counts + (idx[:, m:m + 1] == j_iota).astype(jnp.float32)
        row_max = jnp.max(s, axis=-1, keepdims=True)
        p = counts * jnp.exp(s - row_max)
        denom = jnp.sum(p, axis=-1, keepdims=True)
        weighted.append(_mm(p, xb) / denom)
    agg = x + jnp.concatenate(weighted, axis=0)

    out = _mm(agg, wn_ref[...]) + b_node
    h_nb = _ln(x + jnp.maximum(out, 0.0), g1, be1)

    wv = wv_ref[...]
    v = _mm(_mm(x, wvin_ref[...]) + b_vin, wv[:H]) + _mm(h_nb, wv[H:]) + b_v
    hnode_ref[...] = _ln(h_nb + jnp.maximum(v, 0.0), g2, be2).reshape(
        B, N, H).astype(hnode_ref.dtype)

    we = we_ref[...]
    fw_ref[...] = (_mm(_mm(x, w2_ref[...]) + b12_e, we) + b_e).reshape(
        B, N, H).astype(fw_ref.dtype)
    tw_ref[...] = (_mm(_mm(x, w3_ref[...]) + b3_e, we)).reshape(
        B, N, H).astype(tw_ref.dtype)

    wve = wve_ref[...]
    wce_ref[...] = jnp.concatenate(
        [_mm(w1e_ref[...], we), _mm(wvine_ref[...], wve[:H])],
        axis=1).astype(jnp.bfloat16)
    wvbe_ref[...] = wve[H:].astype(jnp.bfloat16)
    bve_ref[...] = _mm(b_vin_e, wve[:H]) + b_v_e


def _stats_w(H):
    r_io = jax.lax.broadcasted_iota(jnp.int32, (2 * H, 2 * H), 0)
    c_io = jax.lax.broadcasted_iota(jnp.int32, (2 * H, 2 * H), 1)
    mask = (r_io < H) == (c_io < H)
    return jnp.where(mask, jnp.float32(1.0 / H), 0.0).astype(jnp.bfloat16)


def _ln_mxu(y, yb, g, b, sw, H):
    stats = jnp.dot(jnp.concatenate([yb, yb * yb], axis=1), sw,
                    preferred_element_type=jnp.float32)
    mu = stats[:, :H]
    var = stats[:, H:] - mu * mu
    r = jax.lax.rsqrt(var + _LN_EPS)
    return (y - mu) * (r * g) + b


def _edge_kernel(e_ref, fw_ref, tw_ref, wc_ref, wvb_ref, bv_ref,
                 g1_ref, be1_ref, g2_ref, be2_ref, out_ref):
    N, H = tw_ref.shape
    FT = fw_ref.shape[0]

    g1, be1 = g1_ref[...], be1_ref[...]
    g2, be2 = g2_ref[...], be2_ref[...]
    bv = bv_ref[...]

    e32 = e_ref[...]

    u = jnp.dot(e32.astype(jnp.bfloat16), wc_ref[...],
                preferred_element_type=jnp.float32)

    pre = (u[:, :H].reshape(FT, N, H)
           + fw_ref[...][:, None, :] + tw_ref[...][None, :, :])
    h = jnp.maximum(pre, 0.0).reshape(FT * N, H)
    h_nb = _ln(e32 + h, g1, be1)

    v = (u[:, H:] + bv
         + jnp.dot(h_nb.astype(jnp.bfloat16), wvb_ref[...],
                   preferred_element_type=jnp.float32))
    out_ref[...] = _ln(h_nb + jnp.maximum(v, 0.0), g2, be2).astype(
        out_ref.dtype)


def _pick_ft(N, target_rows=8192):
    best = 1
    for ft in range(1, N + 1):
        if N % ft:
            continue
        if ft != N and (ft % 8 != 0 or (ft * N) % 8 != 0) and ft != 1:
            continue
        if ft * N <= target_rows:
            best = ft
    return best


@functools.partial(jax.jit, static_argnames=())
def _run(x, e, neighbor_index,
         W_node, b_node, g1_n, be1_n, W_vin_n, b_vin_n, W_v_n, b_v_n,
         g2_n, be2_n,
         W1_e, b1_e, W2_e, b2_e, W3_e, b3_e, W_e, b_e, g1_e, be1_e,
         W_vin_e, b_vin_e, W_v_e, b_v_e, g2_e, be2_e):
    B, N, H = x.shape
    K = neighbor_index.shape[2]

    wmat = pl.BlockSpec((H, H), lambda b: (0, 0))
    brow = pl.BlockSpec((1, H), lambda b: (0, 0))
    w2mat = pl.BlockSpec((2 * H, H), lambda b: (0, 0))
    h_node, fw, tw, wce, wvbe, bve = pl.pallas_call(
        _node_kernel,
        out_shape=(jax.ShapeDtypeStruct((B, N, H), x.dtype),
                   jax.ShapeDtypeStruct((B, N, H), jnp.float32),
                   jax.ShapeDtypeStruct((B, N, H), jnp.float32),
                   jax.ShapeDtypeStruct((H, 2 * H), jnp.bfloat16),
                   jax.ShapeDtypeStruct((H, H), jnp.bfloat16),
                   jax.ShapeDtypeStruct((1, H), jnp.float32)),
        grid=(1,),
        in_specs=[
            pl.BlockSpec((B, N, H), lambda b: (0, 0, 0)),
            pl.BlockSpec((B, N, K), lambda b: (0, 0, 0)),
            wmat, wmat, w2mat, wmat, wmat, wmat, wmat, wmat, w2mat,
        ] + [brow] * 13,
        out_specs=(
            pl.BlockSpec((B, N, H), lambda b: (0, 0, 0)),
            pl.BlockSpec((B, N, H), lambda b: (0, 0, 0)),
            pl.BlockSpec((B, N, H), lambda b: (0, 0, 0)),
            pl.BlockSpec((H, 2 * H), lambda b: (0, 0)),
            pl.BlockSpec((H, H), lambda b: (0, 0)),
            pl.BlockSpec((1, H), lambda b: (0, 0)),
        ),
        compiler_params=pltpu.CompilerParams(dimension_semantics=("arbitrary",)),
    )(x, neighbor_index, W_node, W_vin_n, W_v_n, W2_e, W3_e, W_e,
      W1_e, W_vin_e, W_v_e,
      b_node, g1_n, be1_n, b_vin_n, b_v_n, g2_n, be2_n,
      b1_e, b2_e, b3_e, b_e, b_vin_e, b_v_e)

    FT = _pick_ft(N)
    TM = FT * N
    e_flat = e.reshape(B, N * N, H)

    out = pl.pallas_call(
        _edge_kernel,
        out_shape=jax.ShapeDtypeStruct((B, N * N, H), e.dtype),
        grid=(B, N // FT),
        in_specs=[
            pl.BlockSpec((None, TM, H), lambda b, t: (b, t, 0)),
            pl.BlockSpec((None, FT, H), lambda b, t: (b, t, 0)),
            pl.BlockSpec((None, N, H), lambda b, t: (b, 0, 0)),
            pl.BlockSpec((H, 2 * H), lambda b, t: (0, 0)),
            pl.BlockSpec((H, H), lambda b, t: (0, 0)),
        ] + [pl.BlockSpec((1, H), lambda b, t: (0, 0))] * 5,
        out_specs=pl.BlockSpec((None, TM, H), lambda b, t: (b, t, 0)),
        compiler_params=pltpu.CompilerParams(
            dimension_semantics=("parallel", "parallel")),
    )(e_flat, fw, tw, wce, wvbe, bve, g1_e, be1_e, g2_e, be2_e)

    return h_node, out.reshape(B, N, N, H)


def kernel(x, e, neighbor_index,
           W_node, b_node, g1_n, be1_n, W_vin_n, b_vin_n, W_v_n, b_v_n,
           g2_n, be2_n,
           W1_e, b1_e, W2_e, b2_e, W3_e, b3_e, W_e, b_e, g1_e, be1_e,
           W_vin_e, b_vin_e, W_v_e, b_v_e, g2_e, be2_e):
    return _run(x, e, neighbor_index,
                W_node, b_node, g1_n, be1_n, W_vin_n, b_vin_n, W_v_n, b_v_n,
                g2_n, be2_n,
                W1_e, b1_e, W2_e, b2_e, W3_e, b3_e, W_e, b_e, g1_e, be1_e,
                W_vin_e, b_vin_e, W_v_e, b_v_e, g2_e, be2_e)


# speedup vs baseline: 1.0334x; 1.0334x over previous
import functools

import jax
import jax.numpy as jnp
from jax.experimental import pallas as pl
from jax.experimental.pallas import tpu as pltpu

_LN_EPS = 1e-5


def _ln(v, g, b):
    mu = jnp.mean(v, axis=-1, keepdims=True)
    var = jnp.mean((v - mu) ** 2, axis=-1, keepdims=True)
    return (v - mu) * jax.lax.rsqrt(var + _LN_EPS) * g + b


def _ln_s(v, gs, b, H):
    mu = jnp.mean(v, axis=-1, keepdims=True)
    d = v - mu
    t = jnp.sum(d * d, axis=-1, keepdims=True)
    return d * jax.lax.rsqrt(t + jnp.float32(H * _LN_EPS)) * gs + b


def _mm(a, b):
    return jnp.dot(a, b, preferred_element_type=jnp.float32)


def _node_kernel(x_ref, idx_ref, wn_ref, wvin_ref, wv_ref, w2_ref, w3_ref,
                 we_ref, w1e_ref, wvine_ref, wve_ref,
                 bn_ref, g1n_ref, be1n_ref, bvinn_ref, bvn_ref, g2n_ref,
                 be2n_ref, b1e_ref, b2e_ref, b3e_ref, be_ref, bvine_b_ref,
                 bve_b_ref,
                 hnode_ref, fw_ref, tw_ref, wce_ref, wvbe_ref, bve_ref):
    B, N, H = x_ref.shape
    K = idx_ref.shape[-1]

    x3 = x_ref[...].astype(jnp.float32)
    idx3 = idx_ref[...]
    x = x3.reshape(B * N, H)

    b_node, g1, be1 = bn_ref[...], g1n_ref[...], be1n_ref[...]
    b_vin, b_v, g2, be2 = bvinn_ref[...], bvn_ref[...], g2n_ref[...], be2n_ref[...]
    b12_e = b1e_ref[...] + b2e_ref[...]
    b3_e, b_e = b3e_ref[...], be_ref[...]
    b_vin_e, b_v_e = bvine_b_ref[...], bve_b_ref[...]

    inv_sqrt_h = jnp.float32(1.0) / jnp.sqrt(jnp.float32(H))
    j_iota = jax.lax.broadcasted_iota(jnp.int32, (N, N), 1)
    weighted = []
    for bi in range(B):
        xb = x3[bi]
        idx = idx3[bi]
        s = jax.lax.dot_general(xb, xb, (((1,), (1,)), ((), ())),
                                preferred_element_type=jnp.float32) * inv_sqrt_h
        counts = jnp.zeros((N, N), jnp.float32)
        for m in range(K):
            counts = counts + (idx[:, m:m + 1] == j_iota).astype(jnp.float32)
        row_max = jnp.max(s, axis=-1, keepdims=True)
        p = counts * jnp.exp(s - row_max)
        denom = jnp.sum(p, axis=-1, keepdims=True)
        weighted.append(_mm(p, xb) / denom)
    agg = x + jnp.concatenate(weighted, axis=0)

    out = _mm(agg, wn_ref[...]) + b_node
    h_nb = _ln(x + jnp.maximum(out, 0.0), g1, be1)

    wv = wv_ref[...]
    v = _mm(_mm(x, wvin_ref[...]) + b_vin, wv[:H]) + _mm(h_nb, wv[H:]) + b_v
    hnode_ref[...] = _ln(h_nb + jnp.maximum(v, 0.0), g2, be2).reshape(
        B, N, H).astype(hnode_ref.dtype)

    we = we_ref[...]
    fw_ref[...] = (_mm(_mm(x, w2_ref[...]) + b12_e, we) + b_e).reshape(
        B, N, H).astype(fw_ref.dtype)
    tw_ref[...] = (_mm(_mm(x, w3_ref[...]) + b3_e, we)).reshape(
        B, N, H).astype(tw_ref.dtype)

    wve = wve_ref[...]
    wce_ref[...] = jnp.concatenate(
        [_mm(w1e_ref[...], we), _mm(wvine_ref[...], wve[:H])],
        axis=1).astype(jnp.bfloat16)
    wvbe_ref[...] = wve[H:].astype(jnp.bfloat16)
    bve_ref[...] = _mm(b_vin_e, wve[:H]) + b_v_e


def _stats_w(H):
    r_io = jax.lax.broadcasted_iota(jnp.int32, (2 * H, 2 * H), 0)
    c_io = jax.lax.broadcasted_iota(jnp.int32, (2 * H, 2 * H), 1)
    mask = (r_io < H) == (c_io < H)
    return jnp.where(mask, jnp.float32(1.0 / H), 0.0).astype(jnp.bfloat16)


def _ln_mxu(y, yb, g, b, sw, H):
    stats = jnp.dot(jnp.concatenate([yb, yb * yb], axis=1), sw,
                    preferred_element_type=jnp.float32)
    mu = stats[:, :H]
    var = stats[:, H:] - mu * mu
    r = jax.lax.rsqrt(var + _LN_EPS)
    return (y - mu) * (r * g) + b


def _edge_kernel(e_ref, fw_ref, tw_ref, wc_ref, wvb_ref, bv_ref,
                 g1_ref, be1_ref, g2_ref, be2_ref, out_ref):
    N, H = tw_ref.shape
    FT = fw_ref.shape[0]

    g1, be1 = g1_ref[...], be1_ref[...]
    g2, be2 = g2_ref[...], be2_ref[...]
    bv = bv_ref[...]

    e32 = e_ref[...]

    u = jnp.dot(e32.astype(jnp.bfloat16), wc_ref[...],
                preferred_element_type=jnp.float32)

    pre = (u[:, :H].reshape(FT, N, H)
           + fw_ref[...][:, None, :] + tw_ref[...][None, :, :])
    h = jnp.maximum(pre, 0.0).reshape(FT * N, H)
    sh = jnp.sqrt(jnp.float32(H))
    h_nb = _ln_s(e32 + h, g1 * sh, be1, H)

    v = (u[:, H:] + bv
         + jnp.dot(h_nb.astype(jnp.bfloat16), wvb_ref[...],
                   preferred_element_type=jnp.float32))
    out_ref[...] = _ln_s(h_nb + jnp.maximum(v, 0.0), g2 * sh, be2, H).astype(
        out_ref.dtype)


def _pick_ft(N, target_rows=8192):
    best = 1
    for ft in range(1, N + 1):
        if N % ft:
            continue
        if ft != N and (ft % 8 != 0 or (ft * N) % 8 != 0) and ft != 1:
            continue
        if ft * N <= target_rows:
            best = ft
    return best


@functools.partial(jax.jit, static_argnames=())
def _run(x, e, neighbor_index,
         W_node, b_node, g1_n, be1_n, W_vin_n, b_vin_n, W_v_n, b_v_n,
         g2_n, be2_n,
         W1_e, b1_e, W2_e, b2_e, W3_e, b3_e, W_e, b_e, g1_e, be1_e,
         W_vin_e, b_vin_e, W_v_e, b_v_e, g2_e, be2_e):
    B, N, H = x.shape
    K = neighbor_index.shape[2]

    wmat = pl.BlockSpec((H, H), lambda b: (0, 0))
    brow = pl.BlockSpec((1, H), lambda b: (0, 0))
    w2mat = pl.BlockSpec((2 * H, H), lambda b: (0, 0))
    h_node, fw, tw, wce, wvbe, bve = pl.pallas_call(
        _node_kernel,
        out_shape=(jax.ShapeDtypeStruct((B, N, H), x.dtype),
                   jax.ShapeDtypeStruct((B, N, H), jnp.float32),
                   jax.ShapeDtypeStruct((B, N, H), jnp.float32),
                   jax.ShapeDtypeStruct((H, 2 * H), jnp.bfloat16),
                   jax.ShapeDtypeStruct((H, H), jnp.bfloat16),
                   jax.ShapeDtypeStruct((1, H), jnp.float32)),
        grid=(1,),
        in_specs=[
            pl.BlockSpec((B, N, H), lambda b: (0, 0, 0)),
            pl.BlockSpec((B, N, K), lambda b: (0, 0, 0)),
            wmat, wmat, w2mat, wmat, wmat, wmat, wmat, wmat, w2mat,
        ] + [brow] * 13,
        out_specs=(
            pl.BlockSpec((B, N, H), lambda b: (0, 0, 0)),
            pl.BlockSpec((B, N, H), lambda b: (0, 0, 0)),
            pl.BlockSpec((B, N, H), lambda b: (0, 0, 0)),
            pl.BlockSpec((H, 2 * H), lambda b: (0, 0)),
            pl.BlockSpec((H, H), lambda b: (0, 0)),
            pl.BlockSpec((1, H), lambda b: (0, 0)),
        ),
        compiler_params=pltpu.CompilerParams(dimension_semantics=("arbitrary",)),
    )(x, neighbor_index, W_node, W_vin_n, W_v_n, W2_e, W3_e, W_e,
      W1_e, W_vin_e, W_v_e,
      b_node, g1_n, be1_n, b_vin_n, b_v_n, g2_n, be2_n,
      b1_e, b2_e, b3_e, b_e, b_vin_e, b_v_e)

    FT = _pick_ft(N)
    TM = FT * N
    e_flat = e.reshape(B, N * N, H)

    out = pl.pallas_call(
        _edge_kernel,
        out_shape=jax.ShapeDtypeStruct((B, N * N, H), e.dtype),
        grid=(B, N // FT),
        in_specs=[
            pl.BlockSpec((None, TM, H), lambda b, t: (b, t, 0)),
            pl.BlockSpec((None, FT, H), lambda b, t: (b, t, 0)),
            pl.BlockSpec((None, N, H), lambda b, t: (b, 0, 0)),
            pl.BlockSpec((H, 2 * H), lambda b, t: (0, 0)),
            pl.BlockSpec((H, H), lambda b, t: (0, 0)),
        ] + [pl.BlockSpec((1, H), lambda b, t: (0, 0))] * 5,
        out_specs=pl.BlockSpec((None, TM, H), lambda b, t: (b, t, 0)),
        compiler_params=pltpu.CompilerParams(
            dimension_semantics=("parallel", "parallel")),
    )(e_flat, fw, tw, wce, wvbe, bve, g1_e, be1_e, g2_e, be2_e)

    return h_node, out.reshape(B, N, N, H)


def kernel(x, e, neighbor_index,
           W_node, b_node, g1_n, be1_n, W_vin_n, b_vin_n, W_v_n, b_v_n,
           g2_n, be2_n,
           W1_e, b1_e, W2_e, b2_e, W3_e, b3_e, W_e, b_e, g1_e, be1_e,
           W_vin_e, b_vin_e, W_v_e, b_v_e, g2_e, be2_e):
    return _run(x, e, neighbor_index,
                W_node, b_node, g1_n, be1_n, W_vin_n, b_vin_n, W_v_n, b_v_n,
                g2_n, be2_n,
                W1_e, b1_e, W2_e, b2_e, W3_e, b3_e, W_e, b_e, g1_e, be1_e,
                W_vin_e, b_vin_e, W_v_e, b_v_e, g2_e, be2_e)


# speedup vs baseline: 1.0617x; 1.0274x over previous
import functools

import jax
import jax.numpy as jnp
from jax.experimental import pallas as pl
from jax.experimental.pallas import tpu as pltpu

_LN_EPS = 1e-5


def _ln(v, g, b):
    mu = jnp.mean(v, axis=-1, keepdims=True)
    var = jnp.mean((v - mu) ** 2, axis=-1, keepdims=True)
    return (v - mu) * jax.lax.rsqrt(var + _LN_EPS) * g + b


def _ln_s(v, gs, b, H):
    mu = jnp.mean(v, axis=-1, keepdims=True)
    d = v - mu
    t = jnp.sum(d * d, axis=-1, keepdims=True)
    return d * jax.lax.rsqrt(t + jnp.float32(H * _LN_EPS)) * gs + b


def _mm(a, b):
    return jnp.dot(a, b, preferred_element_type=jnp.float32)


def _node_kernel(x_ref, idx_ref, wn_ref, wvin_ref, wv_ref, w2_ref, w3_ref,
                 we_ref, w1e_ref, wvine_ref, wve_ref,
                 bn_ref, g1n_ref, be1n_ref, bvinn_ref, bvn_ref, g2n_ref,
                 be2n_ref, b1e_ref, b2e_ref, b3e_ref, be_ref, bvine_b_ref,
                 bve_b_ref, g1e_ref, be1e_ref,
                 hnode_ref, fw_ref, tw_ref, wce_ref, wvbe_ref, bve_ref):
    B, N, H = x_ref.shape
    K = idx_ref.shape[-1]

    x3 = x_ref[...].astype(jnp.float32)
    idx3 = idx_ref[...]
    x = x3.reshape(B * N, H)

    b_node, g1, be1 = bn_ref[...], g1n_ref[...], be1n_ref[...]
    b_vin, b_v, g2, be2 = bvinn_ref[...], bvn_ref[...], g2n_ref[...], be2n_ref[...]
    b12_e = b1e_ref[...] + b2e_ref[...]
    b3_e, b_e = b3e_ref[...], be_ref[...]
    b_vin_e, b_v_e = bvine_b_ref[...], bve_b_ref[...]

    inv_sqrt_h = jnp.float32(1.0) / jnp.sqrt(jnp.float32(H))
    j_iota = jax.lax.broadcasted_iota(jnp.int32, (N, N), 1)
    weighted = []
    for bi in range(B):
        xb = x3[bi]
        idx = idx3[bi]
        s = jax.lax.dot_general(xb, xb, (((1,), (1,)), ((), ())),
                                preferred_element_type=jnp.float32) * inv_sqrt_h
        counts = jnp.zeros((N, N), jnp.float32)
        for m in range(K):
            counts = counts + (idx[:, m:m + 1] == j_iota).astype(jnp.float32)
        row_max = jnp.max(s, axis=-1, keepdims=True)
        p = counts * jnp.exp(s - row_max)
        denom = jnp.sum(p, axis=-1, keepdims=True)
        weighted.append(_mm(p, xb) / denom)
    agg = x + jnp.concatenate(weighted, axis=0)

    out = _mm(agg, wn_ref[...]) + b_node
    h_nb = _ln(x + jnp.maximum(out, 0.0), g1, be1)

    wv = wv_ref[...]
    v = _mm(_mm(x, wvin_ref[...]) + b_vin, wv[:H]) + _mm(h_nb, wv[H:]) + b_v
    hnode_ref[...] = _ln(h_nb + jnp.maximum(v, 0.0), g2, be2).reshape(
        B, N, H).astype(hnode_ref.dtype)

    we = we_ref[...]
    fw_ref[...] = (_mm(_mm(x, w2_ref[...]) + b12_e, we) + b_e).reshape(
        B, N, H).astype(fw_ref.dtype)
    tw_ref[...] = (_mm(_mm(x, w3_ref[...]) + b3_e, we)).reshape(
        B, N, H).astype(tw_ref.dtype)

    wve = wve_ref[...]
    wce_ref[...] = jnp.concatenate(
        [_mm(w1e_ref[...], we), _mm(wvine_ref[...], wve[:H])],
        axis=1).astype(jnp.bfloat16)
    sh = jnp.sqrt(jnp.float32(H))
    wvb = wve[H:]
    wvbe_ref[...] = ((g1e_ref[...] * sh).reshape(H, 1) * wvb).astype(
        jnp.bfloat16)
    bve_ref[...] = _mm(b_vin_e, wve[:H]) + b_v_e + _mm(be1e_ref[...], wvb)


def _stats_w(H):
    r_io = jax.lax.broadcasted_iota(jnp.int32, (2 * H, 2 * H), 0)
    c_io = jax.lax.broadcasted_iota(jnp.int32, (2 * H, 2 * H), 1)
    mask = (r_io < H) == (c_io < H)
    return jnp.where(mask, jnp.float32(1.0 / H), 0.0).astype(jnp.bfloat16)


def _ln_mxu(y, yb, g, b, sw, H):
    stats = jnp.dot(jnp.concatenate([yb, yb * yb], axis=1), sw,
                    preferred_element_type=jnp.float32)
    mu = stats[:, :H]
    var = stats[:, H:] - mu * mu
    r = jax.lax.rsqrt(var + _LN_EPS)
    return (y - mu) * (r * g) + b


def _edge_kernel(e_ref, fw_ref, tw_ref, wc_ref, wvb_ref, bv_ref,
                 g1_ref, be1_ref, g2_ref, be2_ref, out_ref):
    N, H = tw_ref.shape
    FT = fw_ref.shape[0]

    g1, be1 = g1_ref[...], be1_ref[...]
    g2, be2 = g2_ref[...], be2_ref[...]
    bv = bv_ref[...]

    e32 = e_ref[...]

    u = jnp.dot(e32.astype(jnp.bfloat16), wc_ref[...],
                preferred_element_type=jnp.float32)

    pre = (u[:, :H].reshape(FT, N, H)
           + fw_ref[...][:, None, :] + tw_ref[...][None, :, :])
    h = jnp.maximum(pre, 0.0).reshape(FT * N, H)
    sh = jnp.sqrt(jnp.float32(H))
    y1 = e32 + h
    mu1 = jnp.mean(y1, axis=-1, keepdims=True)
    d1 = y1 - mu1
    t1 = jnp.sum(d1 * d1, axis=-1, keepdims=True)
    t = d1 * jax.lax.rsqrt(t1 + jnp.float32(H * _LN_EPS))
    h_nb = t * (g1 * sh) + be1

    v = (u[:, H:] + bv
         + jnp.dot(t.astype(jnp.bfloat16), wvb_ref[...],
                   preferred_element_type=jnp.float32))
    out_ref[...] = _ln_s(h_nb + jnp.maximum(v, 0.0), g2 * sh, be2, H).astype(
        out_ref.dtype)


def _pick_ft(N, target_rows=8192):
    best = 1
    for ft in range(1, N + 1):
        if N % ft:
            continue
        if ft != N and (ft % 8 != 0 or (ft * N) % 8 != 0) and ft != 1:
            continue
        if ft * N <= target_rows:
            best = ft
    return best


@functools.partial(jax.jit, static_argnames=())
def _run(x, e, neighbor_index,
         W_node, b_node, g1_n, be1_n, W_vin_n, b_vin_n, W_v_n, b_v_n,
         g2_n, be2_n,
         W1_e, b1_e, W2_e, b2_e, W3_e, b3_e, W_e, b_e, g1_e, be1_e,
         W_vin_e, b_vin_e, W_v_e, b_v_e, g2_e, be2_e):
    B, N, H = x.shape
    K = neighbor_index.shape[2]

    wmat = pl.BlockSpec((H, H), lambda b: (0, 0))
    brow = pl.BlockSpec((1, H), lambda b: (0, 0))
    w2mat = pl.BlockSpec((2 * H, H), lambda b: (0, 0))
    h_node, fw, tw, wce, wvbe, bve = pl.pallas_call(
        _node_kernel,
        out_shape=(jax.ShapeDtypeStruct((B, N, H), x.dtype),
                   jax.ShapeDtypeStruct((B, N, H), jnp.float32),
                   jax.ShapeDtypeStruct((B, N, H), jnp.float32),
                   jax.ShapeDtypeStruct((H, 2 * H), jnp.bfloat16),
                   jax.ShapeDtypeStruct((H, H), jnp.bfloat16),
                   jax.ShapeDtypeStruct((1, H), jnp.float32)),
        grid=(1,),
        in_specs=[
            pl.BlockSpec((B, N, H), lambda b: (0, 0, 0)),
            pl.BlockSpec((B, N, K), lambda b: (0, 0, 0)),
            wmat, wmat, w2mat, wmat, wmat, wmat, wmat, wmat, w2mat,
        ] + [brow] * 15,
        out_specs=(
            pl.BlockSpec((B, N, H), lambda b: (0, 0, 0)),
            pl.BlockSpec((B, N, H), lambda b: (0, 0, 0)),
            pl.BlockSpec((B, N, H), lambda b: (0, 0, 0)),
            pl.BlockSpec((H, 2 * H), lambda b: (0, 0)),
            pl.BlockSpec((H, H), lambda b: (0, 0)),
            pl.BlockSpec((1, H), lambda b: (0, 0)),
        ),
        compiler_params=pltpu.CompilerParams(dimension_semantics=("arbitrary",)),
    )(x, neighbor_index, W_node, W_vin_n, W_v_n, W2_e, W3_e, W_e,
      W1_e, W_vin_e, W_v_e,
      b_node, g1_n, be1_n, b_vin_n, b_v_n, g2_n, be2_n,
      b1_e, b2_e, b3_e, b_e, b_vin_e, b_v_e, g1_e, be1_e)

    FT = _pick_ft(N)
    TM = FT * N
    e_flat = e.reshape(B, N * N, H)

    out = pl.pallas_call(
        _edge_kernel,
        out_shape=jax.ShapeDtypeStruct((B, N * N, H), e.dtype),
        grid=(B, N // FT),
        in_specs=[
            pl.BlockSpec((None, TM, H), lambda b, t: (b, t, 0)),
            pl.BlockSpec((None, FT, H), lambda b, t: (b, t, 0)),
            pl.BlockSpec((None, N, H), lambda b, t: (b, 0, 0)),
            pl.BlockSpec((H, 2 * H), lambda b, t: (0, 0)),
            pl.BlockSpec((H, H), lambda b, t: (0, 0)),
        ] + [pl.BlockSpec((1, H), lambda b, t: (0, 0))] * 5,
        out_specs=pl.BlockSpec((None, TM, H), lambda b, t: (b, t, 0)),
        compiler_params=pltpu.CompilerParams(
            dimension_semantics=("parallel", "parallel")),
    )(e_flat, fw, tw, wce, wvbe, bve, g1_e, be1_e, g2_e, be2_e)

    return h_node, out.reshape(B, N, N, H)


def kernel(x, e, neighbor_index,
           W_node, b_node, g1_n, be1_n, W_vin_n, b_vin_n, W_v_n, b_v_n,
           g2_n, be2_n,
           W1_e, b1_e, W2_e, b2_e, W3_e, b3_e, W_e, b_e, g1_e, be1_e,
           W_vin_e, b_vin_e, W_v_e, b_v_e, g2_e, be2_e):
    return _run(x, e, neighbor_index,
                W_node, b_node, g1_n, be1_n, W_vin_n, b_vin_n, W_v_n, b_v_n,
                g2_n, be2_n,
                W1_e, b1_e, W2_e, b2_e, W3_e, b3_e, W_e, b_e, g1_e, be1_e,
                W_vin_e, b_vin_e, W_v_e, b_v_e, g2_e, be2_e)
